# baseline (speedup 1.0000x reference)
import jax
import jax.numpy as jnp
from jax import lax
from jax.experimental import pallas as pl
from jax.experimental.pallas import tpu as pltpu
from jax.experimental.pallas import tpu_sc as plsc

NC = 2
NS = 16
L = 16
NW = NC * NS
TOTAL = 32768 * 1024
PER_W = TOTAL // NW
CHUNK = 32768
NCHUNK = PER_W // CHUNK
UNROLL = 32
NACC = 8


def _relu_sum_body(x_hbm, out_hbm, buf0, buf1, accv, sem0, sem1):
    wid = lax.axis_index("s") * NC + lax.axis_index("c")
    base = wid * PER_W
    bufs = (buf0, buf1)
    sems = (sem0, sem1)
    copies = [None, None]

    def start(c):
        b = c % 2
        copies[b] = pltpu.make_async_copy(
            x_hbm.at[pl.ds(base + c * CHUNK, CHUNK)], bufs[b], sems[b])
        copies[b].start()

    start(0)
    accs = tuple(jnp.zeros((L,), jnp.float32) for _ in range(NACC))
    for c in range(NCHUNK):
        b = c % 2
        if c + 1 < NCHUNK:
            start(c + 1)
        copies[b].wait()
        buf = bufs[b]

        def step(i, accs, buf=buf):
            off = i * (UNROLL * L)
            new = list(accs)
            for u in range(UNROLL):
                v = buf[pl.ds(off + u * L, L)]
                new[u % NACC] = new[u % NACC] + jnp.maximum(v, 0.0)
            return tuple(new)

        accs = lax.fori_loop(0, CHUNK // (UNROLL * L), step, accs)

    total = accs[0]
    for a in accs[1:]:
        total = total + a
    accv[...] = total
    pltpu.sync_copy(accv, out_hbm.at[wid])


def kernel(x):
    partials = pl.kernel(
        _relu_sum_body,
        out_type=jax.ShapeDtypeStruct((NW, L), jnp.float32),
        mesh=plsc.VectorSubcoreMesh(core_axis_name="c", subcore_axis_name="s"),
        scratch_types=[
            pltpu.VMEM((CHUNK,), jnp.float32),
            pltpu.VMEM((CHUNK,), jnp.float32),
            pltpu.VMEM((L,), jnp.float32),
            pltpu.SemaphoreType.DMA,
            pltpu.SemaphoreType.DMA,
        ],
    )(x.reshape(-1))
    return jnp.sum(partials)[None]


# speedup vs baseline: 2.5444x; 2.5444x over previous
import jax
import jax.numpy as jnp
from jax import lax
from jax.experimental import pallas as pl
from jax.experimental.pallas import tpu as pltpu
from jax.experimental.pallas import tpu_sc as plsc

NC = 2
NS = 16
L = 16
NW = NC * NS
NROWS = 32768
NCOLS = 1024
ROWS_PER_W = NROWS // NW
CHUNK_R = 32
NCHUNK = ROWS_PER_W // CHUNK_R
NACC = 8


def _relu_sum_body(x_hbm, out_hbm, buf0, buf1, accv, sem0, sem1):
    wid = lax.axis_index("s") * NC + lax.axis_index("c")
    row0 = wid * ROWS_PER_W
    bufs = (buf0, buf1)
    sems = (sem0, sem1)

    def copy(c, b):
        return pltpu.make_async_copy(
            x_hbm.at[pl.ds(row0 + c * CHUNK_R, CHUNK_R), :], bufs[b], sems[b])

    copy(0, 0).start()
    copy(1, 1).start()

    def sum_buf(buf, accs):
        def row_step(r, accs):
            new = list(accs)
            for u in range(NCOLS // L):
                v = buf[r, pl.ds(u * L, L)]
                new[u % NACC] = new[u % NACC] + jnp.maximum(v, 0.0)
            return tuple(new)
        return lax.fori_loop(0, CHUNK_R, row_step, accs)

    def body(c2, accs):
        c = c2 * 2
        copy(c, 0).wait()
        accs = sum_buf(buf0, accs)

        @pl.when(c2 < NCHUNK // 2 - 1)
        def _():
            copy(c + 2, 0).start()

        copy(c + 1, 1).wait()
        accs = sum_buf(buf1, accs)

        @pl.when(c2 < NCHUNK // 2 - 1)
        def _():
            copy(c + 3, 1).start()

        return accs

    accs = lax.fori_loop(
        0, NCHUNK // 2, body,
        tuple(jnp.zeros((L,), jnp.float32) for _ in range(NACC)))

    total = accs[0]
    for a in accs[1:]:
        total = total + a
    accv[...] = total
    pltpu.sync_copy(accv, out_hbm.at[pl.ds(wid * L, L)])


def kernel(x):
    partials = pl.kernel(
        _relu_sum_body,
        out_type=jax.ShapeDtypeStruct((NW * L,), jnp.float32),
        mesh=plsc.VectorSubcoreMesh(core_axis_name="c", subcore_axis_name="s"),
        scratch_types=[
            pltpu.VMEM((CHUNK_R, NCOLS), jnp.float32),
            pltpu.VMEM((CHUNK_R, NCOLS), jnp.float32),
            pltpu.VMEM((L,), jnp.float32),
            pltpu.SemaphoreType.DMA,
            pltpu.SemaphoreType.DMA,
        ],
        compiler_params=pltpu.CompilerParams(use_tc_tiling_on_sc=True),
    )(x)
    return jnp.sum(partials)[None]
